# Initial kernel scaffold; baseline (speedup 1.0000x reference)
import jax
import jax.numpy as jnp
from jax.experimental import pallas as pl


def kernel(x, edge_index, edge_attr, u, W, b):
    raise NotImplementedError("write your pallas kernel here")


# speedup vs baseline: 4.1475x; 4.1475x over previous
import functools

import jax
import jax.numpy as jnp
from jax import lax
from jax.experimental import pallas as pl
from jax.experimental.pallas import tpu as pltpu
from jax.experimental.pallas import tpu_sc as plsc

N = 10000
E = 320000
D_FEAT = 128
D_EDGE = 16

NUM_SC = 2
NUM_TEC = 16
NW = NUM_SC * NUM_TEC

NPAD = 10240
PAD_ROW = N + 100
BLK = 128
NBLK = E // BLK
BLK_PER_W = 80
NBLK_PAD = NW * BLK_PER_W
KI = 8
GROUPS = BLK_PER_W // KI
ZROWS = NPAD // NUM_TEC

@functools.cache
def _make_sc_scatter_add():
    mesh = plsc.VectorSubcoreMesh(core_axis_name="c", subcore_axis_name="s")
    return functools.partial(
        pl.kernel,
        mesh=mesh,
        compiler_params=pltpu.CompilerParams(use_tc_tiling_on_sc=False),
        out_type=jax.ShapeDtypeStruct((NUM_SC, NPAD, D_EDGE), jnp.float32),
        scratch_types=[
            pltpu.VMEM((KI, BLK), jnp.int32),
            pltpu.VMEM((KI * BLK, D_EDGE), jnp.float32),
            pltpu.VMEM_SHARED((NPAD, D_EDGE), jnp.float32),
        ],
    )(_sc_scatter_body)


def _sc_scatter_body(idx_hbm, attr_hbm, zeros_hbm, out_hbm, idx_v, attr_v, acc):
    cid = lax.axis_index("c")
    sid = lax.axis_index("s")
    wid = cid * NUM_TEC + sid

    pltpu.sync_copy(zeros_hbm, acc.at[pl.ds(sid * ZROWS, ZROWS)])
    plsc.subcore_barrier()

    wbase = wid * BLK_PER_W

    def group(t, carry):
        row0 = wbase + t * KI
        pltpu.sync_copy(idx_hbm.at[pl.ds(row0, KI)], idx_v)
        for k in range(KI):
            rc = jnp.minimum(row0 + k, NBLK - 1)
            pltpu.sync_copy(attr_hbm.at[pl.ds(rc * BLK, BLK)],
                            attr_v.at[pl.ds(k * BLK, BLK)])
            pltpu.sync_copy(attr_v.at[pl.ds(k * BLK, BLK)],
                            acc.at[idx_v.at[k]], add=True)
        return carry

    lax.fori_loop(0, GROUPS, group, 0)

    plsc.subcore_barrier()
    pltpu.sync_copy(acc.at[pl.ds(sid * ZROWS, ZROWS)],
                    out_hbm.at[cid, pl.ds(sid * ZROWS, ZROWS)])


ROW_BLK = 2000


def _tc_body(x_ref, p_ref, w_ref, u_ref, b_ref, o_ref):
    agg = p_ref[0] + p_ref[1]
    w_a = w_ref[0:D_EDGE, :]
    w_x = w_ref[D_EDGE:D_EDGE + D_FEAT, :]
    w_u = w_ref[D_EDGE + D_FEAT:, :]
    o_ref[...] = (
        jnp.dot(x_ref[...], w_x, preferred_element_type=jnp.float32)
        + jnp.dot(agg, w_a, preferred_element_type=jnp.float32)
        + u_ref[0] * w_u
        + b_ref[...]
    )


def _tc_node_mlp(x, p, W, u, b):
    grid = (N // ROW_BLK,)
    return pl.pallas_call(
        _tc_body,
        grid=grid,
        in_specs=[
            pl.BlockSpec((ROW_BLK, D_FEAT), lambda i: (i, 0)),
            pl.BlockSpec((NUM_SC, ROW_BLK, D_EDGE), lambda i: (0, i, 0)),
            pl.BlockSpec((D_EDGE + D_FEAT + 1, D_FEAT), lambda i: (0, 0)),
            pl.BlockSpec(memory_space=pltpu.SMEM),
            pl.BlockSpec((1, D_FEAT), lambda i: (0, 0)),
        ],
        out_specs=pl.BlockSpec((ROW_BLK, D_FEAT), lambda i: (i, 0)),
        out_shape=jax.ShapeDtypeStruct((N, D_FEAT), jnp.float32),
    )(x, p, W, u, b)


def kernel(x, edge_index, edge_attr, u, W, b):
    recv = edge_index[1]
    idx2d = jnp.pad(recv, (0, NBLK_PAD * BLK - E),
                    constant_values=PAD_ROW).reshape(NBLK_PAD, BLK)
    zeros = jnp.zeros((ZROWS, D_EDGE), jnp.float32)
    partials = _make_sc_scatter_add()(idx2d, edge_attr, zeros)
    p = partials[:, :N, :]
    return _tc_node_mlp(x, p, W, u.astype(jnp.float32), b.reshape(1, D_FEAT))


# speedup vs baseline: 5.3025x; 1.2785x over previous
import functools

import jax
import jax.numpy as jnp
from jax import lax
from jax.experimental import pallas as pl
from jax.experimental.pallas import tpu as pltpu
from jax.experimental.pallas import tpu_sc as plsc

N = 10000
E = 320000
D_FEAT = 128
D_EDGE = 16

NUM_SC = 2
NUM_TEC = 16
NW = NUM_SC * NUM_TEC

NPAD = 10240
PAD_ROW = N + 100
BLK = 128
NBLK = E // BLK
BLK_PER_W = 80
NBLK_PAD = NW * BLK_PER_W
KI = 4
GROUPS = BLK_PER_W // KI
NBUF = 2
ZROWS = NPAD // NUM_TEC

@functools.cache
def _make_sc_scatter_add():
    mesh = plsc.VectorSubcoreMesh(core_axis_name="c", subcore_axis_name="s")
    return functools.partial(
        pl.kernel,
        mesh=mesh,
        compiler_params=pltpu.CompilerParams(use_tc_tiling_on_sc=False),
        out_type=jax.ShapeDtypeStruct((NUM_SC, NPAD, D_EDGE), jnp.float32),
        scratch_types=[
            pltpu.VMEM((NBUF, KI, BLK), jnp.int32),
            pltpu.VMEM((NBUF, KI * BLK, D_EDGE), jnp.float32),
            pltpu.VMEM_SHARED((NPAD, D_EDGE), jnp.float32),
            pltpu.SemaphoreType.DMA((NBUF,)),
            pltpu.SemaphoreType.DMA,
        ],
    )(_sc_scatter_body)


def _sc_scatter_body(idx_hbm, attr_hbm, zeros_hbm, out_hbm, idx_v, attr_v,
                     acc, sem_in, sem_sc):
    cid = lax.axis_index("c")
    sid = lax.axis_index("s")
    wid = cid * NUM_TEC + sid

    pltpu.sync_copy(zeros_hbm, acc.at[pl.ds(sid * ZROWS, ZROWS)])
    plsc.subcore_barrier()

    wbase = wid * BLK_PER_W

    def start_inputs(g, b):
        row0 = wbase + g * KI
        a0 = jnp.minimum(row0, NBLK - KI) * BLK
        pltpu.async_copy(idx_hbm.at[pl.ds(row0, KI)], idx_v.at[b], sem_in.at[b])
        pltpu.async_copy(attr_hbm.at[pl.ds(a0, KI * BLK)], attr_v.at[b],
                         sem_in.at[b])

    def drain_inputs(b):
        pltpu.make_async_copy(idx_hbm.at[pl.ds(0, KI)], idx_v.at[b],
                              sem_in.at[b]).wait()
        pltpu.make_async_copy(attr_hbm.at[pl.ds(0, KI * BLK)], attr_v.at[b],
                              sem_in.at[b]).wait()

    for b in range(NBUF):
        start_inputs(b, b)

    def outer(i, carry):
        g0 = i * NBUF
        for b in range(NBUF):
            g = g0 + b
            drain_inputs(b)
            descs = [
                pltpu.async_copy(attr_v.at[b, pl.ds(k * BLK, BLK)],
                                 acc.at[idx_v.at[b, k]], sem_sc, add=True)
                for k in range(KI)
            ]
            for d in descs:
                d.wait()

            gn = g + NBUF

            @pl.when(gn < GROUPS)
            def _():
                start_inputs(gn, b)
        return carry

    lax.fori_loop(0, GROUPS // NBUF, outer, 0)

    plsc.subcore_barrier()
    pltpu.sync_copy(acc.at[pl.ds(sid * ZROWS, ZROWS)],
                    out_hbm.at[cid, pl.ds(sid * ZROWS, ZROWS)])


ROW_BLK = 2000


def _tc_body(x_ref, p_ref, w_ref, u_ref, b_ref, o_ref):
    agg = p_ref[0] + p_ref[1]
    w_a = w_ref[0:D_EDGE, :]
    w_x = w_ref[D_EDGE:D_EDGE + D_FEAT, :]
    w_u = w_ref[D_EDGE + D_FEAT:, :]
    o_ref[...] = (
        jnp.dot(x_ref[...], w_x, preferred_element_type=jnp.float32)
        + jnp.dot(agg, w_a, preferred_element_type=jnp.float32)
        + u_ref[0] * w_u
        + b_ref[...]
    )


def _tc_node_mlp(x, p, W, u, b):
    grid = (N // ROW_BLK,)
    return pl.pallas_call(
        _tc_body,
        grid=grid,
        in_specs=[
            pl.BlockSpec((ROW_BLK, D_FEAT), lambda i: (i, 0)),
            pl.BlockSpec((NUM_SC, ROW_BLK, D_EDGE), lambda i: (0, i, 0)),
            pl.BlockSpec((D_EDGE + D_FEAT + 1, D_FEAT), lambda i: (0, 0)),
            pl.BlockSpec(memory_space=pltpu.SMEM),
            pl.BlockSpec((1, D_FEAT), lambda i: (0, 0)),
        ],
        out_specs=pl.BlockSpec((ROW_BLK, D_FEAT), lambda i: (i, 0)),
        out_shape=jax.ShapeDtypeStruct((N, D_FEAT), jnp.float32),
    )(x, p, W, u, b)


def kernel(x, edge_index, edge_attr, u, W, b):
    recv = edge_index[1]
    idx2d = jnp.pad(recv, (0, NBLK_PAD * BLK - E),
                    constant_values=PAD_ROW).reshape(NBLK_PAD, BLK)
    zeros = jnp.zeros((ZROWS, D_EDGE), jnp.float32)
    partials = _make_sc_scatter_add()(idx2d, edge_attr, zeros)
    p = partials[:, :N, :]
    return _tc_node_mlp(x, p, W, u.astype(jnp.float32), b.reshape(1, D_FEAT))
